# baseline (speedup 1.0000x reference)
import functools

import jax
import jax.numpy as jnp
from jax import lax
from jax.experimental import pallas as pl
from jax.experimental.pallas import tpu as pltpu
from jax.experimental.pallas import tpu_sc as plsc

N = 10000
D = 128
LN_EPS = 1e-5

NC = 2
NS = 16
NW = NC * NS
CHUNK = 64
NB = 4
NI = 40
CPT = NB * NI
EPT = CHUNK * CPT
PPT = EPT - 10000
NP = 10112
RPT = NP // NS
NPDEG = 10240
RPTD = NPDEG // NS

_mesh = plsc.VectorSubcoreMesh(core_axis_name="c", subcore_axis_name="s")


@functools.partial(
    pl.kernel,
    out_type=jax.ShapeDtypeStruct((NC * NPDEG,), jnp.float32),
    mesh=_mesh,
    scratch_types=[
        pltpu.VMEM((2, CHUNK), jnp.int32),
        pltpu.VMEM((CHUNK,), jnp.float32),
        pltpu.VMEM((RPTD,), jnp.float32),
        pltpu.VMEM_SHARED((NPDEG,), jnp.float32),
        pltpu.SemaphoreType.DMA,
    ],
)
def _deg_kernel(dst_hbm, deg_out, idx_v, ones_v, zer_v, acc_s, semi):
    c = lax.axis_index("c")
    s = lax.axis_index("s")
    base = (c * NS + s) * CPT
    cp0 = pltpu.async_copy(dst_hbm.at[pl.ds(base * CHUNK, CHUNK)],
                           idx_v.at[0], semi)
    for k in range(CHUNK // 16):
        ones_v[pl.ds(k * 16, 16)] = jnp.ones((16,), jnp.float32)

    def zbody(k, _):
        zer_v[pl.ds(k * 16, 16)] = jnp.zeros((16,), jnp.float32)
        return 0

    lax.fori_loop(0, RPTD // 16, zbody, 0)
    pltpu.sync_copy(zer_v, acc_s.at[pl.ds(s * RPTD, RPTD)])
    cp0.wait()
    plsc.subcore_barrier()

    def body(j, _):
        cur = j % 2
        nxt = 1 - cur

        @pl.when(j + 1 < CPT)
        def _():
            pltpu.async_copy(dst_hbm.at[pl.ds((base + j + 1) * CHUNK, CHUNK)],
                             idx_v.at[nxt], semi)

        pltpu.sync_copy(ones_v, acc_s.at[idx_v.at[cur]], add=True)

        @pl.when(j + 1 < CPT)
        def _():
            pltpu.make_async_copy(
                dst_hbm.at[pl.ds((base + j + 1) * CHUNK, CHUNK)],
                idx_v.at[nxt], semi).wait()

        return 0

    lax.fori_loop(0, CPT, body, 0)
    plsc.subcore_barrier()
    pltpu.sync_copy(acc_s.at[pl.ds(s * RPTD, RPTD)],
                    deg_out.at[pl.ds(c * NPDEG + s * RPTD, RPTD)])


@functools.partial(
    pl.kernel,
    out_type=jax.ShapeDtypeStruct((NC, NP, D), jnp.float32),
    mesh=_mesh,
    scratch_types=(
        [
            pltpu.VMEM((NB, CHUNK, D), jnp.float32),
            pltpu.VMEM((NB, CHUNK), jnp.int32),
            pltpu.VMEM((NB, CHUNK), jnp.int32),
            pltpu.VMEM_SHARED((NP, D), jnp.float32),
        ]
        + [pltpu.SemaphoreType.DMA] * (4 * NB)
    ),
)
def _conv_kernel(y_hbm, src_hbm, dst_hbm, out_hbm, buf, src_i, dst_i, acc_s,
                 *sems):
    semg = sems[0:NB]
    semsc = sems[NB:2 * NB]
    semis = sems[2 * NB:3 * NB]
    semid = sems[3 * NB:4 * NB]
    c = lax.axis_index("c")
    s = lax.axis_index("s")
    base = (c * NS + s) * CPT

    def src_slice(ch):
        return src_hbm.at[pl.ds((base + ch) * CHUNK, CHUNK)]

    def dst_slice(ch):
        return dst_hbm.at[pl.ds((base + ch) * CHUNK, CHUNK)]

    for k in range(NB):
        pltpu.async_copy(src_slice(k), src_i.at[k], semis[k])
        pltpu.async_copy(dst_slice(k), dst_i.at[k], semid[k])

    def zbody(k, _):
        buf[0, k // (D // 16), pl.ds((k % (D // 16)) * 16, 16)] = (
            jnp.zeros((16,), jnp.float32))
        return 0

    lax.fori_loop(0, CHUNK * D // 16, zbody, 0)
    for t in range(RPT // CHUNK):
        pltpu.sync_copy(buf.at[0], acc_s.at[pl.ds(s * RPT + t * CHUNK, CHUNK)])
    pltpu.sync_copy(buf.at[0, pl.ds(0, RPT % CHUNK)],
                    acc_s.at[pl.ds(s * RPT + (RPT // CHUNK) * CHUNK,
                                   RPT % CHUNK)])
    for k in range(NB):
        pltpu.make_async_copy(src_slice(k), src_i.at[k], semis[k]).wait()
    plsc.subcore_barrier()
    for k in range(NB):
        pltpu.async_copy(y_hbm.at[src_i.at[k]], buf.at[k], semg[k])

    def body(i, _):
        a = NB * i
        scs = []
        for k in range(NB):
            pltpu.make_async_copy(dst_slice(a + k), dst_i.at[k],
                                  semid[k]).wait()
            pltpu.make_async_copy(y_hbm.at[src_i.at[k]], buf.at[k],
                                  semg[k]).wait()
            scs.append(pltpu.async_copy(buf.at[k], acc_s.at[dst_i.at[k]],
                                        semsc[k], add=True))
        for k in range(NB):
            @pl.when(a + k + NB < CPT)
            def _(k=k):
                pltpu.async_copy(src_slice(a + k + NB), src_i.at[k], semis[k])
        for k in range(NB):
            scs[k].wait()

            @pl.when(a + k + NB < CPT)
            def _(k=k):
                pltpu.make_async_copy(src_slice(a + k + NB), src_i.at[k],
                                      semis[k]).wait()
                pltpu.async_copy(y_hbm.at[src_i.at[k]], buf.at[k], semg[k])
                pltpu.async_copy(dst_slice(a + k + NB), dst_i.at[k], semid[k])

        return 0

    lax.fori_loop(0, NI, body, 0)
    plsc.subcore_barrier()
    pltpu.sync_copy(acc_s.at[pl.ds(s * RPT, RPT)],
                    out_hbm.at[c, pl.ds(s * RPT, RPT)])


def _ln_mm_scale_body(x_ref, w_ref, sc_ref, bi_ref, deg_ref, b_ref,
                      y_ref, r_ref):
    xv = x_ref[...]
    mu = jnp.mean(xv, axis=1, keepdims=True)
    xc = xv - mu
    var = jnp.mean(xc * xc, axis=1, keepdims=True)
    xn = xc * lax.rsqrt(var + LN_EPS) * sc_ref[...] + bi_ref[...]
    xw = jnp.dot(xn, w_ref[...], preferred_element_type=jnp.float32)
    dinv = lax.rsqrt(deg_ref[...])
    y_ref[...] = xw * dinv
    r_ref[...] = xv + b_ref[...] + dinv * dinv * xw


def _combine_body(p_ref, deg_ref, r_ref, o_ref):
    dinv = lax.rsqrt(deg_ref[...])
    agg = dinv * (p_ref[0] + p_ref[1]) + r_ref[...]
    o_ref[...] = jnp.maximum(agg, 0.0)


_BLK = 1000
_GRID = N // _BLK


def kernel(x, edge_index, edge_attr, h, batch, W, b, ln_scale, ln_bias):
    del edge_attr, batch
    f32 = jnp.float32

    e_per_w = edge_index.shape[1] // NW
    idt = edge_index.dtype
    ar = jnp.arange(PPT, dtype=idt)
    spad = jnp.broadcast_to((ar * 37) % N, (NW, PPT))
    dpad = jnp.broadcast_to(N + ar % (NP - N), (NW, PPT))
    src1 = jnp.concatenate(
        [edge_index[0].reshape(NW, e_per_w), spad], axis=1).reshape(-1)
    dst1 = jnp.concatenate(
        [edge_index[1].reshape(NW, e_per_w), dpad], axis=1).reshape(-1)

    deg_parts = _deg_kernel(dst1)
    dp = deg_parts.reshape(NC, NPDEG)
    deg_col = (dp[0, :N] + dp[1, :N] + 1.0).reshape(N, 1)

    y, r = pl.pallas_call(
        _ln_mm_scale_body,
        grid=(_GRID,),
        in_specs=[
            pl.BlockSpec((_BLK, D), lambda j: (j, 0)),
            pl.BlockSpec((D, D), lambda j: (0, 0)),
            pl.BlockSpec((1, D), lambda j: (0, 0)),
            pl.BlockSpec((1, D), lambda j: (0, 0)),
            pl.BlockSpec((_BLK, 1), lambda j: (j, 0)),
            pl.BlockSpec((1, D), lambda j: (0, 0)),
        ],
        out_specs=[
            pl.BlockSpec((_BLK, D), lambda j: (j, 0)),
            pl.BlockSpec((_BLK, D), lambda j: (j, 0)),
        ],
        out_shape=[
            jax.ShapeDtypeStruct((N, D), f32),
            jax.ShapeDtypeStruct((N, D), f32),
        ],
    )(x, W, ln_scale.reshape(1, D), ln_bias.reshape(1, D), deg_col,
      b.reshape(1, D))

    parts = _conv_kernel(y, src1, dst1)

    out = pl.pallas_call(
        _combine_body,
        grid=(_GRID,),
        in_specs=[
            pl.BlockSpec((NC, _BLK, D), lambda j: (0, j, 0)),
            pl.BlockSpec((_BLK, 1), lambda j: (j, 0)),
            pl.BlockSpec((_BLK, D), lambda j: (j, 0)),
        ],
        out_specs=pl.BlockSpec((_BLK, D), lambda j: (j, 0)),
        out_shape=jax.ShapeDtypeStruct((N, D), f32),
    )(parts, deg_col, r)

    return (out, h)


# speedup vs baseline: 1.1711x; 1.1711x over previous
import functools

import jax
import jax.numpy as jnp
from jax import lax
from jax.experimental import pallas as pl
from jax.experimental.pallas import tpu as pltpu
from jax.experimental.pallas import tpu_sc as plsc

N = 10000
D = 128
LN_EPS = 1e-5

NC = 2
NS = 16
NW = NC * NS
CHUNK = 128
CPT = 80
EPT = CHUNK * CPT
EPAD = EPT * NW
NP = 10240
RPT = NP // NS
NBLK = 8
KB = CPT // NBLK

_mesh = plsc.VectorSubcoreMesh(core_axis_name="c", subcore_axis_name="s")


@functools.partial(
    pl.kernel,
    out_type=jax.ShapeDtypeStruct((NC * NP,), jnp.float32),
    mesh=_mesh,
    scratch_types=[
        pltpu.VMEM((NBLK, KB, CHUNK), jnp.int32),
        pltpu.VMEM((CHUNK,), jnp.float32),
        pltpu.VMEM((RPT,), jnp.float32),
        pltpu.VMEM_SHARED((NP,), jnp.float32),
        pltpu.SemaphoreType.DMA,
    ],
)
def _deg_kernel(dst_hbm, deg_out, dst_v, ones_v, zer_v, acc_s, sem):
    c = lax.axis_index("c")
    s = lax.axis_index("s")
    wid = c * NS + s
    cp = pltpu.async_copy(dst_hbm.at[wid], dst_v, sem)
    for k in range(CHUNK // 16):
        ones_v[pl.ds(k * 16, 16)] = jnp.ones((16,), jnp.float32)

    def zbody(k, _):
        zer_v[pl.ds(k * 16, 16)] = jnp.zeros((16,), jnp.float32)
        return 0

    lax.fori_loop(0, RPT // 16, zbody, 0)
    pltpu.sync_copy(zer_v, acc_s.at[pl.ds(s * RPT, RPT)])
    cp.wait()
    plsc.subcore_barrier()

    def body(j, _):
        pltpu.sync_copy(ones_v, acc_s.at[dst_v.at[j // KB, j % KB]], add=True)
        return 0

    lax.fori_loop(0, CPT, body, 0)
    plsc.subcore_barrier()
    pltpu.sync_copy(acc_s.at[pl.ds(s * RPT, RPT)],
                    deg_out.at[pl.ds(c * NP + s * RPT, RPT)])


HKB = KB // 2


@functools.partial(
    pl.kernel,
    out_type=jax.ShapeDtypeStruct((NC, NP, D), jnp.float32),
    mesh=_mesh,
    scratch_types=[
        pltpu.VMEM((CPT, CHUNK), jnp.int32),
        pltpu.VMEM((2, KB, CHUNK), jnp.int32),
        pltpu.VMEM((CHUNK, D), jnp.float32),
        pltpu.VMEM((CHUNK, D), jnp.float32),
        pltpu.VMEM_SHARED((NP, D), jnp.float32),
        pltpu.SemaphoreType.DMA,
        pltpu.SemaphoreType.DMA,
        pltpu.SemaphoreType.DMA,
        pltpu.SemaphoreType.DMA,
        pltpu.SemaphoreType.DMA,
    ],
)
def _conv_kernel(y_hbm, src_hbm, dst_hbm, out_hbm, src_v, dst_v, buf0, buf1,
                 acc_s, semg0, semg1, sems0, sems1, semi):
    c = lax.axis_index("c")
    s = lax.axis_index("s")
    wid = c * NS + s
    cps = pltpu.async_copy(src_hbm.at[wid], src_v, semi)
    cpd = pltpu.async_copy(dst_hbm.at[wid, 0], dst_v.at[0], semi)

    def zbody(k, _):
        buf0[k // (D // 16), pl.ds((k % (D // 16)) * 16, 16)] = (
            jnp.zeros((16,), jnp.float32))
        return 0

    lax.fori_loop(0, CHUNK * D // 16, zbody, 0)
    for t in range(RPT // CHUNK):
        pltpu.sync_copy(buf0, acc_s.at[pl.ds(s * RPT + t * CHUNK, CHUNK)])
    cps.wait()
    cpd.wait()
    plsc.subcore_barrier()
    pltpu.async_copy(y_hbm.at[src_v.at[0]], buf0, semg0)
    pltpu.async_copy(y_hbm.at[src_v.at[1]], buf1, semg1)

    def body(p, _):
        a = 2 * p
        blk = p // HKB
        cur = blk % 2
        a_loc = (p % HKB) * 2
        at_blk_start = p % HKB == 0

        @pl.when(jnp.logical_and(at_blk_start, blk + 1 < NBLK))
        def _():
            pltpu.async_copy(dst_hbm.at[wid, blk + 1], dst_v.at[1 - cur], semi)

        @pl.when(jnp.logical_and(at_blk_start, blk > 0))
        def _():
            pltpu.make_async_copy(dst_hbm.at[wid, blk],
                                  dst_v.at[cur], semi).wait()

        pltpu.make_async_copy(y_hbm.at[src_v.at[a]], buf0, semg0).wait()
        s0 = pltpu.async_copy(buf0, acc_s.at[dst_v.at[cur, a_loc]], sems0,
                              add=True)
        pltpu.make_async_copy(y_hbm.at[src_v.at[a + 1]], buf1, semg1).wait()
        s1 = pltpu.async_copy(buf1, acc_s.at[dst_v.at[cur, a_loc + 1]], sems1,
                              add=True)
        s0.wait()

        @pl.when(a + 2 < CPT)
        def _():
            pltpu.async_copy(y_hbm.at[src_v.at[a + 2]], buf0, semg0)

        s1.wait()

        @pl.when(a + 3 < CPT)
        def _():
            pltpu.async_copy(y_hbm.at[src_v.at[a + 3]], buf1, semg1)

        return 0

    lax.fori_loop(0, CPT // 2, body, 0)
    plsc.subcore_barrier()
    pltpu.sync_copy(acc_s.at[pl.ds(s * RPT, RPT)],
                    out_hbm.at[c, pl.ds(s * RPT, RPT)])


def _ln_mm_scale_body(x_ref, w_ref, sc_ref, bi_ref, deg_ref, b_ref,
                      y_ref, r_ref):
    xv = x_ref[...]
    mu = jnp.mean(xv, axis=1, keepdims=True)
    xc = xv - mu
    var = jnp.mean(xc * xc, axis=1, keepdims=True)
    xn = xc * lax.rsqrt(var + LN_EPS) * sc_ref[...] + bi_ref[...]
    xw = jnp.dot(xn, w_ref[...], preferred_element_type=jnp.float32)
    dinv = lax.rsqrt(deg_ref[...])
    y_ref[...] = xw * dinv
    r_ref[...] = xv + b_ref[...] + dinv * dinv * xw


def _combine_body(p_ref, deg_ref, r_ref, o_ref):
    dinv = lax.rsqrt(deg_ref[...])
    agg = dinv * (p_ref[0] + p_ref[1]) + r_ref[...]
    o_ref[...] = jnp.maximum(agg, 0.0)


_BLK = 1000
_GRID = N // _BLK


def kernel(x, edge_index, edge_attr, h, batch, W, b, ln_scale, ln_bias):
    del edge_attr, batch
    f32 = jnp.float32

    e = edge_index.shape[1]
    pad = EPAD - e
    ar = jnp.arange(pad, dtype=edge_index.dtype)
    src = jnp.concatenate([edge_index[0], (ar * 37) % N]).reshape(NW, CPT, CHUNK)
    dst = jnp.concatenate([edge_index[1], N + ar % (NP - N)]).reshape(NW, NBLK, KB, CHUNK)

    deg_parts = _deg_kernel(dst)
    dp = deg_parts.reshape(NC, NP)
    deg_col = (dp[0, :N] + dp[1, :N] + 1.0).reshape(N, 1)

    y, r = pl.pallas_call(
        _ln_mm_scale_body,
        grid=(_GRID,),
        in_specs=[
            pl.BlockSpec((_BLK, D), lambda j: (j, 0)),
            pl.BlockSpec((D, D), lambda j: (0, 0)),
            pl.BlockSpec((1, D), lambda j: (0, 0)),
            pl.BlockSpec((1, D), lambda j: (0, 0)),
            pl.BlockSpec((_BLK, 1), lambda j: (j, 0)),
            pl.BlockSpec((1, D), lambda j: (0, 0)),
        ],
        out_specs=[
            pl.BlockSpec((_BLK, D), lambda j: (j, 0)),
            pl.BlockSpec((_BLK, D), lambda j: (j, 0)),
        ],
        out_shape=[
            jax.ShapeDtypeStruct((N, D), f32),
            jax.ShapeDtypeStruct((N, D), f32),
        ],
    )(x, W, ln_scale.reshape(1, D), ln_bias.reshape(1, D), deg_col,
      b.reshape(1, D))

    parts = _conv_kernel(y, src, dst)

    out = pl.pallas_call(
        _combine_body,
        grid=(_GRID,),
        in_specs=[
            pl.BlockSpec((NC, _BLK, D), lambda j: (0, j, 0)),
            pl.BlockSpec((_BLK, 1), lambda j: (j, 0)),
            pl.BlockSpec((_BLK, D), lambda j: (j, 0)),
        ],
        out_specs=pl.BlockSpec((_BLK, D), lambda j: (j, 0)),
        out_shape=jax.ShapeDtypeStruct((N, D), f32),
    )(parts, deg_col, r)

    return (out, h)


# speedup vs baseline: 1.1856x; 1.0124x over previous
import functools

import jax
import jax.numpy as jnp
from jax import lax
from jax.experimental import pallas as pl
from jax.experimental.pallas import tpu as pltpu
from jax.experimental.pallas import tpu_sc as plsc

N = 10000
D = 128
LN_EPS = 1e-5

NC = 2
NS = 16
NW = NC * NS
CHUNK = 128
CPT = 80
EPT = CHUNK * CPT
EPAD = EPT * NW
NP = 10240
RPT = NP // NS
NBLK = 8
KB = CPT // NBLK

_mesh = plsc.VectorSubcoreMesh(core_axis_name="c", subcore_axis_name="s")


@functools.partial(
    pl.kernel,
    out_type=jax.ShapeDtypeStruct((NC * NP,), jnp.float32),
    mesh=_mesh,
    scratch_types=[
        pltpu.VMEM((NBLK, KB, CHUNK), jnp.int32),
        pltpu.VMEM((CHUNK,), jnp.float32),
        pltpu.VMEM((RPT,), jnp.float32),
        pltpu.VMEM_SHARED((NP,), jnp.float32),
        pltpu.SemaphoreType.DMA,
        pltpu.SemaphoreType.DMA,
        pltpu.SemaphoreType.DMA,
    ],
)
def _deg_kernel(dst_hbm, deg_out, dst_v, ones_v, zer_v, acc_s, sem,
                semsc0, semsc1):
    c = lax.axis_index("c")
    s = lax.axis_index("s")
    wid = c * NS + s
    cp = pltpu.async_copy(dst_hbm.at[wid], dst_v, sem)
    for k in range(CHUNK // 16):
        ones_v[pl.ds(k * 16, 16)] = jnp.ones((16,), jnp.float32)

    def zbody(k, _):
        zer_v[pl.ds(k * 16, 16)] = jnp.zeros((16,), jnp.float32)
        return 0

    lax.fori_loop(0, RPT // 16, zbody, 0)
    pltpu.sync_copy(zer_v, acc_s.at[pl.ds(s * RPT, RPT)])
    cp.wait()
    plsc.subcore_barrier()

    def body(p, _):
        j0 = 2 * p
        j1 = j0 + 1

        @pl.when(p > 0)
        def _():
            pltpu.make_async_copy(
                ones_v, acc_s.at[dst_v.at[(j0 - 2) // KB, (j0 - 2) % KB]],
                semsc0).wait()
            pltpu.make_async_copy(
                ones_v, acc_s.at[dst_v.at[(j1 - 2) // KB, (j1 - 2) % KB]],
                semsc1).wait()

        pltpu.async_copy(ones_v, acc_s.at[dst_v.at[j0 // KB, j0 % KB]],
                         semsc0, add=True)
        pltpu.async_copy(ones_v, acc_s.at[dst_v.at[j1 // KB, j1 % KB]],
                         semsc1, add=True)
        return 0

    lax.fori_loop(0, CPT // 2, body, 0)
    pltpu.make_async_copy(
        ones_v, acc_s.at[dst_v.at[(CPT - 2) // KB, (CPT - 2) % KB]],
        semsc0).wait()
    pltpu.make_async_copy(
        ones_v, acc_s.at[dst_v.at[(CPT - 1) // KB, (CPT - 1) % KB]],
        semsc1).wait()
    plsc.subcore_barrier()
    pltpu.sync_copy(acc_s.at[pl.ds(s * RPT, RPT)],
                    deg_out.at[pl.ds(c * NP + s * RPT, RPT)])


HKB = KB // 2


@functools.partial(
    pl.kernel,
    out_type=jax.ShapeDtypeStruct((NC, NP, D), jnp.float32),
    mesh=_mesh,
    scratch_types=[
        pltpu.VMEM((CPT, CHUNK), jnp.int32),
        pltpu.VMEM((2, KB, CHUNK), jnp.int32),
        pltpu.VMEM((CHUNK, D), jnp.float32),
        pltpu.VMEM((CHUNK, D), jnp.float32),
        pltpu.VMEM_SHARED((NP, D), jnp.float32),
        pltpu.SemaphoreType.DMA,
        pltpu.SemaphoreType.DMA,
        pltpu.SemaphoreType.DMA,
        pltpu.SemaphoreType.DMA,
        pltpu.SemaphoreType.DMA,
    ],
)
def _conv_kernel(y_hbm, src_hbm, dst_hbm, out_hbm, src_v, dst_v, buf0, buf1,
                 acc_s, semg0, semg1, sems0, sems1, semi):
    c = lax.axis_index("c")
    s = lax.axis_index("s")
    wid = c * NS + s
    cps = pltpu.async_copy(src_hbm.at[wid], src_v, semi)
    cpd = pltpu.async_copy(dst_hbm.at[wid, 0], dst_v.at[0], semi)

    def zbody(k, _):
        buf0[k // (D // 16), pl.ds((k % (D // 16)) * 16, 16)] = (
            jnp.zeros((16,), jnp.float32))
        return 0

    lax.fori_loop(0, CHUNK * D // 16, zbody, 0)
    for t in range(RPT // CHUNK):
        pltpu.sync_copy(buf0, acc_s.at[pl.ds(s * RPT + t * CHUNK, CHUNK)])
    cps.wait()
    cpd.wait()
    plsc.subcore_barrier()
    pltpu.async_copy(y_hbm.at[src_v.at[0]], buf0, semg0)
    pltpu.async_copy(y_hbm.at[src_v.at[1]], buf1, semg1)

    def body(p, _):
        a = 2 * p
        blk = p // HKB
        cur = blk % 2
        a_loc = (p % HKB) * 2
        at_blk_start = p % HKB == 0

        @pl.when(jnp.logical_and(at_blk_start, blk + 1 < NBLK))
        def _():
            pltpu.async_copy(dst_hbm.at[wid, blk + 1], dst_v.at[1 - cur], semi)

        @pl.when(jnp.logical_and(at_blk_start, blk > 0))
        def _():
            pltpu.make_async_copy(dst_hbm.at[wid, blk],
                                  dst_v.at[cur], semi).wait()

        pltpu.make_async_copy(y_hbm.at[src_v.at[a]], buf0, semg0).wait()
        s0 = pltpu.async_copy(buf0, acc_s.at[dst_v.at[cur, a_loc]], sems0,
                              add=True)
        pltpu.make_async_copy(y_hbm.at[src_v.at[a + 1]], buf1, semg1).wait()
        s1 = pltpu.async_copy(buf1, acc_s.at[dst_v.at[cur, a_loc + 1]], sems1,
                              add=True)
        s0.wait()

        @pl.when(a + 2 < CPT)
        def _():
            pltpu.async_copy(y_hbm.at[src_v.at[a + 2]], buf0, semg0)

        s1.wait()

        @pl.when(a + 3 < CPT)
        def _():
            pltpu.async_copy(y_hbm.at[src_v.at[a + 3]], buf1, semg1)

        return 0

    lax.fori_loop(0, CPT // 2, body, 0)
    plsc.subcore_barrier()
    pltpu.sync_copy(acc_s.at[pl.ds(s * RPT, RPT)],
                    out_hbm.at[c, pl.ds(s * RPT, RPT)])


def _ln_mm_scale_body(x_ref, w_ref, sc_ref, bi_ref, deg_ref, b_ref,
                      y_ref, r_ref):
    xv = x_ref[...]
    mu = jnp.mean(xv, axis=1, keepdims=True)
    xc = xv - mu
    var = jnp.mean(xc * xc, axis=1, keepdims=True)
    xn = xc * lax.rsqrt(var + LN_EPS) * sc_ref[...] + bi_ref[...]
    xw = jnp.dot(xn, w_ref[...], preferred_element_type=jnp.float32)
    dinv = lax.rsqrt(deg_ref[...])
    y_ref[...] = xw * dinv
    r_ref[...] = xv + b_ref[...] + dinv * dinv * xw


def _combine_body(p_ref, deg_ref, r_ref, o_ref):
    dinv = lax.rsqrt(deg_ref[...])
    agg = dinv * (p_ref[0] + p_ref[1]) + r_ref[...]
    o_ref[...] = jnp.maximum(agg, 0.0)


_BLK = 1000
_GRID = N // _BLK


def kernel(x, edge_index, edge_attr, h, batch, W, b, ln_scale, ln_bias):
    del edge_attr, batch
    f32 = jnp.float32

    e = edge_index.shape[1]
    pad = EPAD - e
    ar = jnp.arange(pad, dtype=edge_index.dtype)
    src = jnp.concatenate([edge_index[0], (ar * 37) % N]).reshape(NW, CPT, CHUNK)
    dst = jnp.concatenate([edge_index[1], N + ar % (NP - N)]).reshape(NW, NBLK, KB, CHUNK)

    deg_parts = _deg_kernel(dst)
    dp = deg_parts.reshape(NC, NP)
    deg_col = (dp[0, :N] + dp[1, :N] + 1.0).reshape(N, 1)

    y, r = pl.pallas_call(
        _ln_mm_scale_body,
        grid=(_GRID,),
        in_specs=[
            pl.BlockSpec((_BLK, D), lambda j: (j, 0)),
            pl.BlockSpec((D, D), lambda j: (0, 0)),
            pl.BlockSpec((1, D), lambda j: (0, 0)),
            pl.BlockSpec((1, D), lambda j: (0, 0)),
            pl.BlockSpec((_BLK, 1), lambda j: (j, 0)),
            pl.BlockSpec((1, D), lambda j: (0, 0)),
        ],
        out_specs=[
            pl.BlockSpec((_BLK, D), lambda j: (j, 0)),
            pl.BlockSpec((_BLK, D), lambda j: (j, 0)),
        ],
        out_shape=[
            jax.ShapeDtypeStruct((N, D), f32),
            jax.ShapeDtypeStruct((N, D), f32),
        ],
    )(x, W, ln_scale.reshape(1, D), ln_bias.reshape(1, D), deg_col,
      b.reshape(1, D))

    parts = _conv_kernel(y, src, dst)

    out = pl.pallas_call(
        _combine_body,
        grid=(_GRID,),
        in_specs=[
            pl.BlockSpec((NC, _BLK, D), lambda j: (0, j, 0)),
            pl.BlockSpec((_BLK, 1), lambda j: (j, 0)),
            pl.BlockSpec((_BLK, D), lambda j: (j, 0)),
        ],
        out_specs=pl.BlockSpec((_BLK, D), lambda j: (j, 0)),
        out_shape=jax.ShapeDtypeStruct((N, D), f32),
    )(parts, deg_col, r)

    return (out, h)
